# baseline (speedup 1.0000x reference)
import jax
from jax.experimental import pallas as pl

_ROWS = 224 * 224
_COLS = 768
_BLOCK_ROWS = 3136


def _copy_block(x_ref, o_ref):
    o_ref[...] = x_ref[...]


def kernel(input):
    x = input.reshape(_COLS, _ROWS).T
    out = pl.pallas_call(
        _copy_block,
        grid=(_ROWS // _BLOCK_ROWS,),
        in_specs=[pl.BlockSpec((_BLOCK_ROWS, _COLS), lambda i: (i, 0))],
        out_specs=pl.BlockSpec((_BLOCK_ROWS, _COLS), lambda i: (i, 0)),
        out_shape=jax.ShapeDtypeStruct((_ROWS, _COLS), x.dtype),
    )(x)
    return out.T.reshape(input.shape)


# speedup vs baseline: 1.0037x; 1.0037x over previous
import jax
from jax.experimental import pallas as pl

_ROWS = 224 * 224
_COLS = 768
_BLOCK_ROWS = 3584


def _copy_block(x_ref, o_ref):
    o_ref[...] = x_ref[...]


def kernel(input):
    x = input.reshape(_COLS, _ROWS).T
    out = pl.pallas_call(
        _copy_block,
        grid=(_ROWS // _BLOCK_ROWS,),
        in_specs=[pl.BlockSpec((_BLOCK_ROWS, _COLS), lambda i: (i, 0))],
        out_specs=pl.BlockSpec((_BLOCK_ROWS, _COLS), lambda i: (i, 0)),
        out_shape=jax.ShapeDtypeStruct((_ROWS, _COLS), x.dtype),
    )(x)
    return out.T.reshape(input.shape)
